# baseline (speedup 1.0000x reference)
import functools

import jax
import jax.numpy as jnp
from jax import lax
from jax.experimental import pallas as pl
from jax.experimental.pallas import tpu as pltpu
from jax.experimental.pallas import tpu_sc as plsc

_VOCAB = 1000000
_D = 32
_B = 16384
_H = 50
_HP = 64
_NW = 32
_ROWS_W = _B // _NW
_K = 4
_NJ = _ROWS_W // _K
_L = 16

_mesh = plsc.VectorSubcoreMesh(core_axis_name="c", subcore_axis_name="s")


@functools.partial(
    pl.kernel,
    mesh=_mesh,
    out_type=jax.ShapeDtypeStruct((_B, _H, _D), jnp.float32),
    scratch_types=(
        [pltpu.VMEM((_ROWS_W, _HP), jnp.int32)]
        + [pltpu.VMEM((_HP,), jnp.int32) for _ in range(_K)]
        + [pltpu.VMEM((_HP, 128), jnp.float32) for _ in range(_K)]
        + [pltpu.VMEM((_H, _D), jnp.float32) for _ in range(_K)]
        + [pltpu.SemaphoreType.DMA for _ in range(2 * _K)]
    ),
    compiler_params=pltpu.CompilerParams(needs_layout_passes=False),
)
def _emb_sum(data_hbm, w_hbm, p_hbm, s_hbm, out_hbm, idx_v, *scratch):
    divs = scratch[:_K]
    accs = scratch[_K:2 * _K]
    obufs = scratch[2 * _K:3 * _K]
    gsems = scratch[3 * _K:4 * _K]
    osems = scratch[4 * _K:]

    wid = lax.axis_index("s") * 2 + lax.axis_index("c")
    base = wid * _ROWS_W
    pltpu.sync_copy(data_hbm.at[pl.ds(base, _ROWS_W)], idx_v)

    iota = lax.iota(jnp.int32, _L)

    def build_div(k, r):
        for c0 in range(0, _HP, _L):
            divs[k][pl.ds(c0, _L)] = lax.shift_right_logical(
                idx_v[r, pl.ds(c0, _L)], 2)

    def fire_w(k):
        return pltpu.async_copy(w_hbm.at[divs[k]], accs[k], gsems[k])

    def fire_ps(k):
        pltpu.async_copy(p_hbm.at[divs[k]], accs[k], gsems[k], add=True)
        pltpu.async_copy(s_hbm.at[divs[k]], accs[k], gsems[k], add=True)

    def extract(k, r):
        for c0 in range(0, _H, _L):
            n = min(_L, _H - c0)
            rowv = iota + c0
            rem16 = lax.shift_left(idx_v[r, pl.ds(c0, _L)] & 3, 5)
            mask = (iota < n) if n < _L else None
            for d in range(_D):
                val = plsc.load_gather(accs[k], [rowv, rem16 + d])
                dv = jnp.full((_L,), d, jnp.int32)
                if mask is None:
                    plsc.store_scatter(obufs[k], [rowv, dv], val)
                else:
                    plsc.store_scatter(obufs[k], [rowv, dv], val, mask=mask)

    def fire_out(k, r):
        return pltpu.async_copy(obufs[k], out_hbm.at[base + r], osems[k])

    def drain(k, n):
        for _ in range(n):
            pltpu.make_async_copy(
                w_hbm.at[pl.ds(0, _HP)], accs[k], gsems[k]).wait()

    descs = []
    for k in range(_K):
        build_div(k, k)
        descs.append(fire_w(k))
    for k in range(_K):
        descs[k].wait()
        fire_ps(k)

    def body(j, _):
        outs = []
        for k in range(_K):
            drain(k, 2)
            extract(k, (j - 1) * _K + k)
            outs.append(fire_out(k, (j - 1) * _K + k))
        wds = []
        for k in range(_K):
            outs[k].wait()
            build_div(k, j * _K + k)
            wds.append(fire_w(k))
        for k in range(_K):
            wds[k].wait()
            fire_ps(k)
        return ()

    lax.fori_loop(1, _NJ, body, ())

    outs = []
    for k in range(_K):
        drain(k, 2)
        extract(k, (_NJ - 1) * _K + k)
        outs.append(fire_out(k, (_NJ - 1) * _K + k))
    for k in range(_K):
        outs[k].wait()


def kernel(data, W_word, W_pre, W_suf):
    data_p = jnp.pad(data, ((0, 0), (0, _HP - _H)))
    w128 = W_word.reshape(_VOCAB // 4, 128)
    p128 = W_pre.reshape(_VOCAB // 4, 128)
    s128 = W_suf.reshape(_VOCAB // 4, 128)
    return _emb_sum(data_p, w128, p128, s128)


# speedup vs baseline: 1.0019x; 1.0019x over previous
import functools

import jax
import jax.numpy as jnp
from jax import lax
from jax.experimental import pallas as pl
from jax.experimental.pallas import tpu as pltpu
from jax.experimental.pallas import tpu_sc as plsc

_VOCAB = 1000000
_D = 32
_B = 16384
_H = 50
_HP = 64
_NW = 32
_ROWS_W = _B // _NW
_K = 4
_NJ = _ROWS_W // _K
_L = 16

_mesh = plsc.VectorSubcoreMesh(core_axis_name="c", subcore_axis_name="s")


@functools.partial(
    pl.kernel,
    mesh=_mesh,
    out_type=jax.ShapeDtypeStruct((_B, _H, _D), jnp.float32),
    scratch_types=(
        [pltpu.VMEM((_ROWS_W, _HP), jnp.int32)]
        + [pltpu.VMEM((_HP,), jnp.int32) for _ in range(_K)]
        + [pltpu.VMEM((_HP, 128), jnp.float32) for _ in range(_K)]
        + [pltpu.VMEM((_H, _D), jnp.float32) for _ in range(_K)]
        + [pltpu.SemaphoreType.DMA for _ in range(2 * _K)]
    ),
    compiler_params=pltpu.CompilerParams(needs_layout_passes=False),
)
def _emb_sum(data_hbm, w_hbm, p_hbm, s_hbm, out_hbm, idx_v, *scratch):
    divs = scratch[:_K]
    accs = scratch[_K:2 * _K]
    obufs = scratch[2 * _K:3 * _K]
    gsems = scratch[3 * _K:4 * _K]
    osems = scratch[4 * _K:]

    wid = lax.axis_index("s") * 2 + lax.axis_index("c")
    base = wid * _ROWS_W
    pltpu.sync_copy(data_hbm.at[pl.ds(base, _ROWS_W)], idx_v)

    iota = lax.iota(jnp.int32, _L)

    def build_div(k, r):
        for c0 in range(0, _HP, _L):
            divs[k][pl.ds(c0, _L)] = lax.shift_right_logical(
                idx_v[r, pl.ds(c0, _L)], 2)

    def fire_w(k):
        return pltpu.async_copy(w_hbm.at[divs[k]], accs[k], gsems[k])

    def fire_ps(k):
        pltpu.async_copy(p_hbm.at[divs[k]], accs[k], gsems[k], add=True)
        pltpu.async_copy(s_hbm.at[divs[k]], accs[k], gsems[k], add=True)

    zeros = jnp.zeros((_L,), jnp.int32)

    def extract(k, r):
        for c0 in range(0, _H, _L):
            n = min(_L, _H - c0)
            rowv = iota + c0
            rem16 = lax.shift_left(idx_v[r, pl.ds(c0, _L)] & 3, 5)
            mask = (iota < n) if n < _L else None

            def dbody(d, _):
                val = plsc.load_gather(accs[k], [rowv, rem16 + d])
                dv = zeros + d
                if mask is None:
                    plsc.store_scatter(obufs[k], [rowv, dv], val)
                else:
                    plsc.store_scatter(obufs[k], [rowv, dv], val, mask=mask)
                return ()

            lax.fori_loop(0, _D, dbody, (), unroll=4)

    def fire_out(k, r):
        return pltpu.async_copy(obufs[k], out_hbm.at[base + r], osems[k])

    def drain(k, n):
        for _ in range(n):
            pltpu.make_async_copy(
                w_hbm.at[pl.ds(0, _HP)], accs[k], gsems[k]).wait()

    descs = []
    for k in range(_K):
        build_div(k, k)
        descs.append(fire_w(k))
    for k in range(_K):
        descs[k].wait()
        fire_ps(k)

    def body(j, _):
        outs = []
        for k in range(_K):
            drain(k, 2)
            extract(k, (j - 1) * _K + k)
            outs.append(fire_out(k, (j - 1) * _K + k))
        wds = []
        for k in range(_K):
            outs[k].wait()
            build_div(k, j * _K + k)
            wds.append(fire_w(k))
        for k in range(_K):
            wds[k].wait()
            fire_ps(k)
        return ()

    lax.fori_loop(1, _NJ, body, ())

    outs = []
    for k in range(_K):
        drain(k, 2)
        extract(k, (_NJ - 1) * _K + k)
        outs.append(fire_out(k, (_NJ - 1) * _K + k))
    for k in range(_K):
        outs[k].wait()


def kernel(data, W_word, W_pre, W_suf):
    data_p = jnp.pad(data, ((0, 0), (0, _HP - _H)))
    w128 = W_word.reshape(_VOCAB // 4, 128)
    p128 = W_pre.reshape(_VOCAB // 4, 128)
    s128 = W_suf.reshape(_VOCAB // 4, 128)
    return _emb_sum(data_p, w128, p128, s128)


# speedup vs baseline: 6.2400x; 6.2282x over previous
import functools

import jax
import jax.numpy as jnp
from jax import lax
from jax.experimental import pallas as pl
from jax.experimental.pallas import tpu as pltpu
from jax.experimental.pallas import tpu_sc as plsc

_VOCAB = 1000000
_D = 32
_B = 16384
_H = 50
_NW = 32
_ROWS_W = _B // _NW
_K = 8
_NJ = _ROWS_W // _K

_mesh = plsc.VectorSubcoreMesh(core_axis_name="c", subcore_axis_name="s")


@functools.partial(
    pl.kernel,
    mesh=_mesh,
    out_type=jax.ShapeDtypeStruct((_B, _H, _D), jnp.float32),
    scratch_types=(
        [pltpu.VMEM((_ROWS_W, _H), jnp.int32)]
        + [pltpu.VMEM((_H, _D), jnp.float32) for _ in range(_K)]
        + [pltpu.SemaphoreType.DMA for _ in range(2 * _K)]
    ),
    compiler_params=pltpu.CompilerParams(use_tc_tiling_on_sc=False),
)
def _emb_sum(data_hbm, w_hbm, p_hbm, s_hbm, out_hbm, idx_v, *scratch):
    bufs = scratch[:_K]
    gsems = scratch[_K:2 * _K]
    osems = scratch[2 * _K:]

    wid = lax.axis_index("s") * 2 + lax.axis_index("c")
    base = wid * _ROWS_W
    pltpu.sync_copy(data_hbm.at[pl.ds(base, _ROWS_W)], idx_v)

    def fire_w(k, r):
        return pltpu.async_copy(w_hbm.at[idx_v.at[r]], bufs[k], gsems[k])

    def fire_ps(k, r):
        pltpu.async_copy(p_hbm.at[idx_v.at[r]], bufs[k], gsems[k], add=True)
        pltpu.async_copy(s_hbm.at[idx_v.at[r]], bufs[k], gsems[k], add=True)

    def fire_out(k, r):
        return pltpu.async_copy(bufs[k], out_hbm.at[base + r], osems[k])

    def drain(k, n):
        for _ in range(n):
            pltpu.make_async_copy(
                w_hbm.at[pl.ds(0, _H)], bufs[k], gsems[k]).wait()

    descs = [fire_w(k, k) for k in range(_K)]
    for k in range(_K):
        descs[k].wait()
        fire_ps(k, k)

    def body(j, _):
        outs = []
        for k in range(_K):
            drain(k, 2)
            outs.append(fire_out(k, (j - 1) * _K + k))
        wds = []
        for k in range(_K):
            outs[k].wait()
            wds.append(fire_w(k, j * _K + k))
        for k in range(_K):
            wds[k].wait()
            fire_ps(k, j * _K + k)
        return ()

    lax.fori_loop(1, _NJ, body, ())

    outs = []
    for k in range(_K):
        drain(k, 2)
        outs.append(fire_out(k, (_NJ - 1) * _K + k))
    for k in range(_K):
        outs[k].wait()


def kernel(data, W_word, W_pre, W_suf):
    return _emb_sum(data, W_word, W_pre, W_suf)
